# Initial kernel scaffold; baseline (speedup 1.0000x reference)
import jax
import jax.numpy as jnp
from jax.experimental import pallas as pl


def kernel(s, r, w, s_table, r_table, w_table):
    raise NotImplementedError("write your pallas kernel here")


# speedup vs baseline: 1.2220x; 1.2220x over previous
import functools

import jax
import jax.numpy as jnp
from jax import lax
from jax.experimental import pallas as pl
from jax.experimental.pallas import tpu as pltpu
from jax.experimental.pallas import tpu_sc as plsc

_K = 128
_L = 16


def _build(batch, kdim):
    info = plsc.get_sparse_core_info()
    nc, ns = info.num_cores, info.num_subcores
    nw = nc * ns
    b_per_w = batch // nw
    chunk = 128
    n_chunks = b_per_w // chunk
    n_groups = chunk // _L
    n_j = kdim // _L

    mesh = plsc.VectorSubcoreMesh(core_axis_name="c", subcore_axis_name="s")

    @functools.partial(
        pl.kernel,
        mesh=mesh,
        out_type=jax.ShapeDtypeStruct((batch,), jnp.float32),
        compiler_params=pltpu.CompilerParams(needs_layout_passes=False),
        scratch_types=[
            pltpu.VMEM((chunk,), jnp.int32),
            pltpu.VMEM((chunk,), jnp.int32),
            pltpu.VMEM((chunk,), jnp.int32),
            pltpu.VMEM((chunk, kdim), jnp.float32),
            pltpu.VMEM((chunk, kdim), jnp.float32),
            pltpu.VMEM((chunk, kdim), jnp.float32),
            pltpu.VMEM((chunk,), jnp.float32),
            pltpu.VMEM((_L * _L,), jnp.float32),
            pltpu.SemaphoreType.DMA,
        ],
    )
    def sc_kernel(s_hbm, r_hbm, w_hbm, st_hbm, rt_hbm, wt_hbm, out_hbm,
                  idx_s, idx_r, idx_w, rows_s, rows_r, rows_w,
                  out_buf, tr_buf, sem):
        wid = lax.axis_index("s") * nc + lax.axis_index("c")
        base_w = wid * b_per_w

        def chunk_body(ci, carry):
            base = base_w + ci * chunk
            pltpu.sync_copy(s_hbm.at[pl.ds(base, chunk)], idx_s)
            pltpu.sync_copy(r_hbm.at[pl.ds(base, chunk)], idx_r)
            pltpu.sync_copy(w_hbm.at[pl.ds(base, chunk)], idx_w)
            cs = pltpu.async_copy(st_hbm.at[idx_s], rows_s, sem)
            cr = pltpu.async_copy(rt_hbm.at[idx_r], rows_r, sem)
            cw = pltpu.async_copy(wt_hbm.at[idx_w], rows_w, sem)
            cs.wait()
            cr.wait()
            cw.wait()

            def group_body(g, carry2):
                lane_ids = lax.iota(jnp.int32, _L)
                tot = jnp.zeros((_L,), jnp.float32)
                for e in range(_L):
                    i = g * _L + e
                    acc = jnp.zeros((_L,), jnp.float32)
                    for j in range(n_j):
                        sv = rows_s[i, pl.ds(j * _L, _L)]
                        rv = rows_r[i, pl.ds(j * _L, _L)]
                        wv = rows_w[i, pl.ds(j * _L, _L)]
                        acc = acc + (sv + rv) * wv
                    tot = jnp.where(lane_ids == e, jnp.sum(acc), tot)
                prob = 1.0 / (1.0 + jnp.exp(-tot))
                out_buf[pl.ds(g * _L, _L)] = prob
                return carry2

            lax.fori_loop(0, n_groups, group_body, 0)
            pltpu.sync_copy(out_buf, out_hbm.at[pl.ds(base, chunk)])
            return carry

        lax.fori_loop(0, n_chunks, chunk_body, 0)

    return sc_kernel


@jax.jit
def kernel(s, r, w, s_table, r_table, w_table):
    batch = s.shape[0]
    fn = _build(batch, s_table.shape[1])
    s32 = s.reshape(-1).astype(jnp.int32)
    r32 = r.reshape(-1).astype(jnp.int32)
    w32 = w.reshape(-1).astype(jnp.int32)
    return fn(s32, r32, w32, s_table, r_table, w_table)


# speedup vs baseline: 1.5576x; 1.2747x over previous
import functools

import jax
import jax.numpy as jnp
from jax import lax
from jax.experimental import pallas as pl
from jax.experimental.pallas import tpu as pltpu
from jax.experimental.pallas import tpu_sc as plsc

_K = 128
_L = 16
_TR = 17


def _build(batch, kdim):
    info = plsc.get_sparse_core_info()
    nc, ns = info.num_cores, info.num_subcores
    nw = nc * ns
    b_per_w = batch // nw
    chunk = 128
    n_chunks = b_per_w // chunk
    n_groups = chunk // _L
    n_j = kdim // _L

    mesh = plsc.VectorSubcoreMesh(core_axis_name="c", subcore_axis_name="s")

    @functools.partial(
        pl.kernel,
        mesh=mesh,
        out_type=jax.ShapeDtypeStruct((batch,), jnp.float32),
        compiler_params=pltpu.CompilerParams(needs_layout_passes=False),
        scratch_types=[
            pltpu.VMEM((2, chunk), jnp.int32),
            pltpu.VMEM((2, chunk), jnp.int32),
            pltpu.VMEM((2, chunk), jnp.int32),
            pltpu.VMEM((2, chunk, kdim), jnp.float32),
            pltpu.VMEM((2, chunk, kdim), jnp.float32),
            pltpu.VMEM((2, chunk, kdim), jnp.float32),
            pltpu.VMEM((chunk,), jnp.float32),
            pltpu.VMEM(((_L - 1) * _TR + _L,), jnp.float32),
            pltpu.SemaphoreType.DMA,
            pltpu.SemaphoreType.DMA,
        ],
    )
    def sc_kernel(s_hbm, r_hbm, w_hbm, st_hbm, rt_hbm, wt_hbm, out_hbm,
                  idx_s, idx_r, idx_w, rows_s, rows_r, rows_w,
                  out_buf, tr_buf, sem0, sem1):
        wid = lax.axis_index("s") * nc + lax.axis_index("c")
        base_w = wid * b_per_w
        sems = [sem0, sem1]

        def fire(ci, slot):
            base = base_w + ci * chunk
            pltpu.sync_copy(s_hbm.at[pl.ds(base, chunk)], idx_s.at[slot])
            pltpu.sync_copy(r_hbm.at[pl.ds(base, chunk)], idx_r.at[slot])
            pltpu.sync_copy(w_hbm.at[pl.ds(base, chunk)], idx_w.at[slot])
            return (
                pltpu.async_copy(st_hbm.at[idx_s.at[slot]], rows_s.at[slot],
                                 sems[slot]),
                pltpu.async_copy(rt_hbm.at[idx_r.at[slot]], rows_r.at[slot],
                                 sems[slot]),
                pltpu.async_copy(wt_hbm.at[idx_w.at[slot]], rows_w.at[slot],
                                 sems[slot]),
            )

        def compute(ci, slot):
            rs, rr, rw = rows_s.at[slot], rows_r.at[slot], rows_w.at[slot]

            def group_body(g, carry):
                lane_ids = lax.iota(jnp.int32, _L)
                tr_idx = lane_ids * _TR
                for e in range(_L):
                    i = g * _L + e
                    acc = jnp.zeros((_L,), jnp.float32)
                    for j in range(n_j):
                        sv = rs[i, pl.ds(j * _L, _L)]
                        rv = rr[i, pl.ds(j * _L, _L)]
                        wv = rw[i, pl.ds(j * _L, _L)]
                        acc = acc + (sv + rv) * wv
                    plsc.store_scatter(tr_buf, [tr_idx + e], acc)
                tot = jnp.zeros((_L,), jnp.float32)
                for l in range(_L):
                    tot = tot + tr_buf[pl.ds(l * _TR, _L)]
                prob = 1.0 / (1.0 + jnp.exp(-tot))
                out_buf[pl.ds(g * _L, _L)] = prob
                return carry

            lax.fori_loop(0, n_groups, group_body, 0)
            base = base_w + ci * chunk
            pltpu.sync_copy(out_buf, out_hbm.at[pl.ds(base, chunk)])

        pending = fire(0, 0)
        for ci in range(n_chunks):
            nxt = None
            if ci + 1 < n_chunks:
                nxt = fire(ci + 1, (ci + 1) % 2)
            for c in pending:
                c.wait()
            compute(ci, ci % 2)
            pending = nxt

    return sc_kernel


@jax.jit
def kernel(s, r, w, s_table, r_table, w_table):
    batch = s.shape[0]
    fn = _build(batch, s_table.shape[1])
    s32 = s.reshape(-1).astype(jnp.int32)
    r32 = r.reshape(-1).astype(jnp.int32)
    w32 = w.reshape(-1).astype(jnp.int32)
    return fn(s32, r32, w32, s_table, r_table, w_table)


# speedup vs baseline: 1.6198x; 1.0399x over previous
import functools

import jax
import jax.numpy as jnp
from jax import lax
from jax.experimental import pallas as pl
from jax.experimental.pallas import tpu as pltpu
from jax.experimental.pallas import tpu_sc as plsc

_K = 128
_L = 16
_TR = 17


def _build(batch, kdim):
    info = plsc.get_sparse_core_info()
    nc, ns = info.num_cores, info.num_subcores
    nw = nc * ns
    b_per_w = batch // nw
    chunk = 128
    n_chunks = b_per_w // chunk
    n_groups = chunk // _L
    n_j = kdim // _L

    mesh = plsc.VectorSubcoreMesh(core_axis_name="c", subcore_axis_name="s")

    @functools.partial(
        pl.kernel,
        mesh=mesh,
        out_type=jax.ShapeDtypeStruct((batch,), jnp.float32),
        compiler_params=pltpu.CompilerParams(needs_layout_passes=False),
        scratch_types=[
            pltpu.VMEM((2, chunk), jnp.int32),
            pltpu.VMEM((2, chunk), jnp.int32),
            pltpu.VMEM((2, chunk), jnp.int32),
            pltpu.VMEM((2, chunk, kdim), jnp.float32),
            pltpu.VMEM((2, chunk, kdim), jnp.float32),
            pltpu.VMEM((2, chunk, kdim), jnp.float32),
            pltpu.VMEM((chunk,), jnp.float32),
            pltpu.VMEM(((_L - 1) * _TR + _L,), jnp.float32),
            pltpu.SemaphoreType.DMA,
            pltpu.SemaphoreType.DMA,
        ],
    )
    def sc_kernel(s_hbm, r_hbm, w_hbm, st_hbm, rt_hbm, wt_hbm, out_hbm,
                  idx_s, idx_r, idx_w, rows_s, rows_r, rows_w,
                  out_buf, tr_buf, sem0, sem1):
        wid = lax.axis_index("s") * nc + lax.axis_index("c")
        base_w = wid * b_per_w
        sems = [sem0, sem1]

        def fire(ci, slot):
            base = base_w + ci * chunk
            pltpu.sync_copy(s_hbm.at[pl.ds(base, chunk)], idx_s.at[slot])
            pltpu.sync_copy(r_hbm.at[pl.ds(base, chunk)], idx_r.at[slot])
            pltpu.sync_copy(w_hbm.at[pl.ds(base, chunk)], idx_w.at[slot])
            return (
                pltpu.async_copy(st_hbm.at[idx_s.at[slot]], rows_s.at[slot],
                                 sems[slot]),
                pltpu.async_copy(rt_hbm.at[idx_r.at[slot]], rows_r.at[slot],
                                 sems[slot]),
                pltpu.async_copy(wt_hbm.at[idx_w.at[slot]], rows_w.at[slot],
                                 sems[slot]),
            )

        def compute(ci, slot):
            rs, rr, rw = rows_s.at[slot], rows_r.at[slot], rows_w.at[slot]

            def group_body(g, carry):
                lane_ids = lax.iota(jnp.int32, _L)
                tr_idx = lane_ids * _TR
                for e0 in range(0, _L, 4):
                    accs = []
                    for e in range(e0, e0 + 4):
                        i = g * _L + e
                        parts = []
                        for j in range(n_j):
                            sv = rs[i, pl.ds(j * _L, _L)]
                            rv = rr[i, pl.ds(j * _L, _L)]
                            wv = rw[i, pl.ds(j * _L, _L)]
                            parts.append((sv + rv) * wv)
                        while len(parts) > 1:
                            parts = [parts[k] + parts[k + 1]
                                     for k in range(0, len(parts), 2)]
                        accs.append(parts[0])
                    for e in range(e0, e0 + 4):
                        plsc.store_scatter(tr_buf, [tr_idx + e], accs[e - e0])
                sums = [tr_buf[pl.ds(l * _TR, _L)] for l in range(_L)]
                while len(sums) > 1:
                    sums = [sums[k] + sums[k + 1]
                            for k in range(0, len(sums), 2)]
                prob = 1.0 / (1.0 + jnp.exp(-sums[0]))
                out_buf[pl.ds(g * _L, _L)] = prob
                return carry

            lax.fori_loop(0, n_groups, group_body, 0)
            base = base_w + ci * chunk
            pltpu.sync_copy(out_buf, out_hbm.at[pl.ds(base, chunk)])

        pending = fire(0, 0)
        for ci in range(n_chunks):
            nxt = None
            if ci + 1 < n_chunks:
                nxt = fire(ci + 1, (ci + 1) % 2)
            for c in pending:
                c.wait()
            compute(ci, ci % 2)
            pending = nxt

    return sc_kernel


@jax.jit
def kernel(s, r, w, s_table, r_table, w_table):
    batch = s.shape[0]
    fn = _build(batch, s_table.shape[1])
    s32 = s.reshape(-1).astype(jnp.int32)
    r32 = r.reshape(-1).astype(jnp.int32)
    w32 = w.reshape(-1).astype(jnp.int32)
    return fn(s32, r32, w32, s_table, r_table, w_table)


# speedup vs baseline: 1.7366x; 1.0721x over previous
import functools

import jax
import jax.numpy as jnp
from jax import lax
from jax.experimental import pallas as pl
from jax.experimental.pallas import tpu as pltpu
from jax.experimental.pallas import tpu_sc as plsc

_K = 128
_L = 16
_TR = 17


def _build(batch, kdim):
    info = plsc.get_sparse_core_info()
    nc, ns = info.num_cores, info.num_subcores
    nw = nc * ns
    b_per_w = batch // nw
    chunk = 128
    n_chunks = b_per_w // chunk
    n_groups = chunk // _L
    n_j = kdim // _L

    mesh = plsc.VectorSubcoreMesh(core_axis_name="c", subcore_axis_name="s")

    @functools.partial(
        pl.kernel,
        mesh=mesh,
        out_type=jax.ShapeDtypeStruct((batch,), jnp.float32),
        compiler_params=pltpu.CompilerParams(needs_layout_passes=False,
                                             skip_device_barrier=True),
        scratch_types=[
            pltpu.VMEM((2, chunk), jnp.int32),
            pltpu.VMEM((2, chunk), jnp.int32),
            pltpu.VMEM((2, chunk), jnp.int32),
            pltpu.VMEM((2, chunk, kdim), jnp.float32),
            pltpu.VMEM((2, chunk, kdim), jnp.float32),
            pltpu.VMEM((2, chunk, kdim), jnp.float32),
            pltpu.VMEM((2, chunk), jnp.float32),
            pltpu.VMEM(((_L - 1) * _TR + _L,), jnp.float32),
            pltpu.SemaphoreType.DMA,
            pltpu.SemaphoreType.DMA,
            pltpu.SemaphoreType.DMA,
            pltpu.SemaphoreType.DMA,
            pltpu.SemaphoreType.DMA,
            pltpu.SemaphoreType.DMA,
        ],
    )
    def sc_kernel(s_hbm, r_hbm, w_hbm, st_hbm, rt_hbm, wt_hbm, out_hbm,
                  idx_s, idx_r, idx_w, rows_s, rows_r, rows_w,
                  out_buf, tr_buf, semr0, semr1, semi0, semi1, semo0, semo1):
        wid = lax.axis_index("s") * nc + lax.axis_index("c")
        base_w = wid * b_per_w
        semr = [semr0, semr1]
        semi = [semi0, semi1]
        semo = [semo0, semo1]

        def idx_load(ci, slot):
            base = base_w + ci * chunk
            return (
                pltpu.async_copy(s_hbm.at[pl.ds(base, chunk)],
                                 idx_s.at[slot], semi[slot]),
                pltpu.async_copy(r_hbm.at[pl.ds(base, chunk)],
                                 idx_r.at[slot], semi[slot]),
                pltpu.async_copy(w_hbm.at[pl.ds(base, chunk)],
                                 idx_w.at[slot], semi[slot]),
            )

        def fire_rows(slot):
            return (
                pltpu.async_copy(st_hbm.at[idx_s.at[slot]], rows_s.at[slot],
                                 semr[slot]),
                pltpu.async_copy(rt_hbm.at[idx_r.at[slot]], rows_r.at[slot],
                                 semr[slot]),
                pltpu.async_copy(wt_hbm.at[idx_w.at[slot]], rows_w.at[slot],
                                 semr[slot]),
            )

        def compute(slot):
            rs, rr, rw = rows_s.at[slot], rows_r.at[slot], rows_w.at[slot]
            ob = out_buf.at[slot]

            def group_body(g, carry):
                lane_ids = lax.iota(jnp.int32, _L)
                tr_idx = lane_ids * _TR
                for e0 in range(0, _L, 4):
                    accs = []
                    for e in range(e0, e0 + 4):
                        i = g * _L + e
                        parts = []
                        for j in range(n_j):
                            sv = rs[i, pl.ds(j * _L, _L)]
                            rv = rr[i, pl.ds(j * _L, _L)]
                            wv = rw[i, pl.ds(j * _L, _L)]
                            parts.append((sv + rv) * wv)
                        while len(parts) > 1:
                            parts = [parts[k] + parts[k + 1]
                                     for k in range(0, len(parts), 2)]
                        accs.append(parts[0])
                    for e in range(e0, e0 + 4):
                        plsc.store_scatter(tr_buf, [tr_idx + e], accs[e - e0])
                sums = [tr_buf[pl.ds(l * _TR, _L)] for l in range(_L)]
                while len(sums) > 1:
                    sums = [sums[k] + sums[k + 1]
                            for k in range(0, len(sums), 2)]
                prob = 1.0 / (1.0 + jnp.exp(-sums[0]))
                ob[pl.ds(g * _L, _L)] = prob
                return carry

            lax.fori_loop(0, n_groups, group_body, 0)

        pend_idx = {0: idx_load(0, 0)}
        for c in pend_idx[0]:
            c.wait()
        pend_rows = {0: fire_rows(0)}
        if n_chunks > 1:
            pend_idx[1] = idx_load(1, 1)
        pend_out = {}
        for ci in range(n_chunks):
            slot = ci % 2
            if ci + 1 < n_chunks:
                for c in pend_idx[ci + 1]:
                    c.wait()
            for c in pend_rows[ci]:
                c.wait()
            if ci + 2 < n_chunks:
                pend_idx[ci + 2] = idx_load(ci + 2, slot)
            if ci + 1 < n_chunks:
                pend_rows[ci + 1] = fire_rows((ci + 1) % 2)
            if ci - 2 >= 0:
                pend_out[ci - 2].wait()
            compute(slot)
            base = base_w + ci * chunk
            pend_out[ci] = pltpu.async_copy(
                out_buf.at[slot], out_hbm.at[pl.ds(base, chunk)], semo[slot])
        for ci in (n_chunks - 2, n_chunks - 1):
            if ci >= 0:
                pend_out[ci].wait()

    return sc_kernel


@jax.jit
def kernel(s, r, w, s_table, r_table, w_table):
    batch = s.shape[0]
    fn = _build(batch, s_table.shape[1])
    s32 = s.reshape(-1).astype(jnp.int32)
    r32 = r.reshape(-1).astype(jnp.int32)
    w32 = w.reshape(-1).astype(jnp.int32)
    return fn(s32, r32, w32, s_table, r_table, w_table)
